# baseline (speedup 1.0000x reference)
import functools

import jax
import jax.numpy as jnp
from jax import lax
from jax.experimental import pallas as pl
from jax.experimental.pallas import tpu as pltpu
from jax.experimental.pallas import tpu_sc as plsc


def _iota16():
    return lax.iota(jnp.int32, 16)


def _dispatch(s, fn):
    pl.when(s == 0)(lambda: fn(0))
    pl.when(s == 1)(lambda: fn(1))


@functools.lru_cache(maxsize=None)
def _make_relayout(V, E, L, B):
    info = plsc.get_sparse_core_info()
    NC, NS = info.num_cores, info.num_subcores
    NW = NC * NS
    assert E == 64 and L % 8 == 0 and B % 128 == 0
    CW = 384
    NT = V // CW
    TAIL = V - NT * CW
    PER_W = NT // NW
    EXTRA = NT - PER_W * NW
    LT = L // 8
    EP = E + 1
    mesh = plsc.VectorSubcoreMesh(core_axis_name="c", subcore_axis_name="s")

    @functools.partial(
        pl.kernel,
        mesh=mesh,
        out_type=(
            jax.ShapeDtypeStruct((V * E,), jnp.float32),
            jax.ShapeDtypeStruct((L * B,), jnp.int32),
        ),
        scratch_types=[
            pltpu.VMEM((2 * E, CW), jnp.float32),
            pltpu.VMEM((2 * CW * E,), jnp.float32),
            pltpu.VMEM((8, 1024), jnp.int32),
            pltpu.VMEM((TAIL * E,), jnp.float32),
            pltpu.SemaphoreType.DMA,
            pltpu.SemaphoreType.DMA,
            pltpu.SemaphoreType.DMA,
            pltpu.SemaphoreType.DMA,
        ],
        compiler_params=pltpu.CompilerParams(use_tc_tiling_on_sc=True,
                                             needs_layout_passes=False),
    )
    def body(wt_hbm, idxt_hbm, wtail_hbm, wflat_hbm, idxflat_hbm,
             inbuf, outbuf, idxbuf, tailbuf, si0, si1, so0, so1):
        wid = lax.axis_index("s") * NC + lax.axis_index("c")
        t0 = wid * PER_W + jnp.minimum(wid, EXTRA)
        n_t = PER_W + (wid < EXTRA).astype(jnp.int32)
        sis = (si0, si1)
        sos = (so0, so1)
        iota = _iota16()
        vbase = [iota + 16 * j for j in range(8)]

        @pl.when(wid < LT)
        def _():
            for c in range(B // 1024):
                pltpu.sync_copy(
                    idxt_hbm.at[pl.ds(8 * wid, 8), pl.ds(1024 * c, 1024)],
                    idxbuf)
                for j in range(8):
                    pltpu.sync_copy(
                        idxbuf.at[j],
                        idxflat_hbm.at[pl.ds((8 * wid + j) * B + 1024 * c,
                                             1024)])

        def start_in(i, s):
            pltpu.async_copy(wt_hbm.at[:, pl.ds((t0 + i) * CW, CW)],
                             inbuf.at[pl.ds(s * E, E)], sis[s])

        def wait_in(s):
            pltpu.make_async_copy(wt_hbm.at[:, pl.ds(0, CW)],
                                  inbuf.at[pl.ds(s * E, E)], sis[s]).wait()

        def start_out(i, s):
            pltpu.async_copy(outbuf.at[pl.ds(s * CW * E, CW * E)],
                             wflat_hbm.at[pl.ds((t0 + i) * CW * E, CW * E)],
                             sos[s])

        def wait_out(s):
            pltpu.make_async_copy(outbuf.at[pl.ds(s * CW * E, CW * E)],
                                  wflat_hbm.at[pl.ds(0, CW * E)], sos[s]).wait()

        ROT = [(iota + d) % 16 for d in range(16)]
        IE = iota * E

        def transpose_block(s):
            srow = s * E
            soff = s * (CW * E)

            @plsc.parallel_loop(0, CW, step=16)
            def _(v0):
                col = v0 + iota
                basev = IE + (v0 * E + soff)
                for q in range(4):
                    baseq = basev + 16 * q
                    for h in range(2):
                        ds_ = range(8 * h, 8 * h + 8)
                        xs = [plsc.load_gather(
                            inbuf, [srow + 16 * q + ROT[d], col]) for d in ds_]
                        for x, d in zip(xs, ds_):
                            plsc.store_scatter(outbuf, [baseq + ROT[d]], x)

        start_in(0, 0)
        start_in(1, 1)

        def block_loop(i, carry):
            s = lax.rem(i, 2)
            _dispatch(s, wait_in)
            pl.when(i >= 2)(lambda: _dispatch(s, wait_out))
            transpose_block(s)
            _dispatch(s, lambda sp: start_out(i, sp))
            pl.when(i + 2 < n_t)(
                lambda: _dispatch(s, lambda sp: start_in(i + 2, sp)))
            return carry

        lax.fori_loop(0, n_t, block_loop, 0)
        wait_out(0)
        wait_out(1)

        if TAIL:
            @pl.when(wid == EXTRA)
            def _():
                pltpu.sync_copy(wtail_hbm, tailbuf)
                pltpu.sync_copy(tailbuf,
                                wflat_hbm.at[pl.ds((V - TAIL) * E, TAIL * E)])

    return body


@functools.lru_cache(maxsize=None)
def _make_gather(V, E, L, B):
    info = plsc.get_sparse_core_info()
    NC, NS = info.num_cores, info.num_subcores
    NW = NC * NS
    CB = B // 128
    NBLK = L * CB
    PER_W = NBLK // NW
    EP = E + 1
    assert PER_W * NW == NBLK and PER_W % 2 == 0
    mesh = plsc.VectorSubcoreMesh(core_axis_name="c", subcore_axis_name="s")

    @functools.partial(
        pl.kernel,
        mesh=mesh,
        out_type=jax.ShapeDtypeStruct((L, E // 8, B // 128, 8, 128),
                                      jnp.float32),
        scratch_types=[
            pltpu.VMEM((2, 128), jnp.int32),
            pltpu.VMEM((2 * 128, E), jnp.float32),
            pltpu.VMEM((128, EP), jnp.float32),
            pltpu.VMEM((2, 8, 8, 128), jnp.float32),
            pltpu.SemaphoreType.DMA,
            pltpu.SemaphoreType.DMA,
            pltpu.SemaphoreType.DMA,
            pltpu.SemaphoreType.DMA,
            pltpu.SemaphoreType.DMA,
            pltpu.SemaphoreType.DMA,
        ],
        compiler_params=pltpu.CompilerParams(use_tc_tiling_on_sc=False,
                                             needs_layout_passes=False),
    )
    def body(w_hbm, idx_hbm, out_hbm, idxv, rowsv, skew, outb,
             mi0, mi1, mg0, mg1, mo0, mo1):
        wid = lax.axis_index("s") * NC + lax.axis_index("c")
        q0 = wid * PER_W
        mis = (mi0, mi1)
        mgs = (mg0, mg1)
        mos = (mo0, mo1)
        iota = _iota16()
        vb = [16 * j + iota for j in range(8)]

        def start_idx(i, s):
            q = q0 + i
            off = lax.div(q, CB) * B + lax.rem(q, CB) * 128
            pltpu.async_copy(idx_hbm.at[pl.ds(off, 128)], idxv.at[s], mis[s])

        def wait_idx(s):
            pltpu.make_async_copy(idx_hbm.at[pl.ds(0, 128)], idxv.at[s],
                                  mis[s]).wait()

        def start_gather(s):
            pltpu.async_copy(w_hbm.at[idxv.at[s]],
                             rowsv.at[pl.ds(s * 128, 128)], mgs[s])

        def wait_gather(s):
            pltpu.make_async_copy(w_hbm.at[idxv.at[s]],
                                  rowsv.at[pl.ds(s * 128, 128)], mgs[s]).wait()

        def start_out(i, s):
            q = q0 + i
            l = lax.div(q, CB)
            c = lax.rem(q, CB)
            pltpu.async_copy(outb.at[s], out_hbm.at[l, :, c], mos[s])

        def wait_out(s):
            pltpu.make_async_copy(outb.at[s], out_hbm.at[0, :, 0],
                                  mos[s]).wait()

        def transpose_block(s):
            rbase = s * 128

            @plsc.parallel_loop(0, 128, step=2, unroll=2)
            def _(k0):
                for dk in range(2):
                    k = k0 + dk
                    for q in range(4):
                        x = rowsv[rbase + k, pl.ds(16 * q, 16)]
                        skew[k, pl.ds(16 * q, 16)] = x

            @plsc.parallel_loop(0, E, unroll=4)
            def _(e):
                esplat = jnp.full((16,), e, jnp.int32)
                r = lax.div(e, 8)
                e8 = lax.rem(e, 8)
                for j in range(8):
                    x = plsc.load_gather(skew, [vb[j], esplat])
                    outb[s, r, e8, pl.ds(j * 16, 16)] = x

        start_idx(0, 0)
        start_idx(1, 1)
        wait_idx(0)
        start_gather(0)
        wait_idx(1)
        start_gather(1)

        def block_loop(i, carry):
            s = lax.rem(i, 2)
            _dispatch(s, wait_gather)
            pl.when(i + 2 < PER_W)(
                lambda: _dispatch(s, lambda sp: start_idx(i + 2, sp)))
            pl.when(i >= 2)(lambda: _dispatch(s, wait_out))
            transpose_block(s)
            _dispatch(s, lambda sp: start_out(i, sp))
            def _refill(sp):
                wait_idx(sp)
                start_gather(sp)

            pl.when(i + 2 < PER_W)(lambda: _dispatch(s, _refill))
            return carry

        lax.fori_loop(0, PER_W, block_loop, 0)
        wait_out(0)
        wait_out(1)

    return body


def kernel(batch_word_indexes, word_embedding):
    B, L = batch_word_indexes.shape
    V, E = word_embedding.shape
    idx_t = jnp.transpose(batch_word_indexes)
    w_t = jnp.transpose(word_embedding)
    n_tail = V % 128
    w_tail = word_embedding[V - n_tail:, :].reshape(-1)
    w_flat, idx_flat = _make_relayout(V, E, L, B)(w_t, idx_t, w_tail)
    w2 = w_flat.reshape(V, E)
    out5 = _make_gather(V, E, L, B)(w2, idx_flat)
    return jnp.transpose(out5, (2, 4, 0, 1, 3)).reshape(B, L, E)


# speedup vs baseline: 1.5179x; 1.5179x over previous
import functools

import jax
import jax.numpy as jnp
from jax import lax
from jax.experimental import pallas as pl
from jax.experimental.pallas import tpu as pltpu
from jax.experimental.pallas import tpu_sc as plsc


def _iota16():
    return lax.iota(jnp.int32, 16)


def _dispatch(s, fn):
    pl.when(s == 0)(lambda: fn(0))
    pl.when(s == 1)(lambda: fn(1))


@functools.lru_cache(maxsize=None)
def _make_relayout(V, E, L, B):
    info = plsc.get_sparse_core_info()
    NC, NS = info.num_cores, info.num_subcores
    NW = NC * NS
    assert E == 64 and L % 8 == 0 and B % 128 == 0
    CW = 256
    NT = V // CW
    TAIL = V - NT * CW
    PER_W = NT // NW
    EXTRA = NT - PER_W * NW
    LT = L // 8
    EP = E + 1
    mesh = plsc.VectorSubcoreMesh(core_axis_name="c", subcore_axis_name="s")

    @functools.partial(
        pl.kernel,
        mesh=mesh,
        out_type=(
            jax.ShapeDtypeStruct((V * E,), jnp.float32),
            jax.ShapeDtypeStruct((L * B,), jnp.int32),
        ),
        scratch_types=[
            pltpu.VMEM((2 * E, CW), jnp.float32),
            pltpu.VMEM((2 * CW * E,), jnp.float32),
            pltpu.VMEM((8, B), jnp.int32),
            pltpu.VMEM((TAIL * E,), jnp.float32),
            pltpu.SemaphoreType.DMA,
            pltpu.SemaphoreType.DMA,
            pltpu.SemaphoreType.DMA,
            pltpu.SemaphoreType.DMA,
        ],
        compiler_params=pltpu.CompilerParams(use_tc_tiling_on_sc=True,
                                             needs_layout_passes=False),
    )
    def body(wt_hbm, idxt_hbm, wtail_hbm, wflat_hbm, idxflat_hbm,
             inbuf, outbuf, idxbuf, tailbuf, si0, si1, so0, so1):
        wid = lax.axis_index("s") * NC + lax.axis_index("c")
        t0 = wid * PER_W + jnp.minimum(wid, EXTRA)
        n_t = PER_W + (wid < EXTRA).astype(jnp.int32)
        sis = (si0, si1)
        sos = (so0, so1)
        iota = _iota16()
        vbase = [iota + 16 * j for j in range(8)]

        @pl.when(wid < LT)
        def _():
            pltpu.sync_copy(idxt_hbm.at[pl.ds(8 * wid, 8), :], idxbuf)
            for j in range(8):
                pltpu.sync_copy(idxbuf.at[j],
                                idxflat_hbm.at[pl.ds((8 * wid + j) * B, B)])

        def start_in(i, s):
            pltpu.async_copy(wt_hbm.at[:, pl.ds((t0 + i) * CW, CW)],
                             inbuf.at[pl.ds(s * E, E)], sis[s])

        def wait_in(s):
            pltpu.make_async_copy(wt_hbm.at[:, pl.ds(0, CW)],
                                  inbuf.at[pl.ds(s * E, E)], sis[s]).wait()

        def start_out(i, s):
            pltpu.async_copy(outbuf.at[pl.ds(s * CW * E, CW * E)],
                             wflat_hbm.at[pl.ds((t0 + i) * CW * E, CW * E)],
                             sos[s])

        def wait_out(s):
            pltpu.make_async_copy(outbuf.at[pl.ds(s * CW * E, CW * E)],
                                  wflat_hbm.at[pl.ds(0, CW * E)], sos[s]).wait()

        ROT = [(iota + d) % 16 for d in range(16)]
        IE = iota * E

        def transpose_block(s):
            srow = s * E
            soff = s * (CW * E)

            @plsc.parallel_loop(0, CW, step=16)
            def _(v0):
                col = v0 + iota
                basev = IE + (v0 * E + soff)
                for q in range(4):
                    baseq = basev + 16 * q
                    for h in range(2):
                        ds_ = range(8 * h, 8 * h + 8)
                        xs = [plsc.load_gather(
                            inbuf, [srow + 16 * q + ROT[d], col]) for d in ds_]
                        for x, d in zip(xs, ds_):
                            plsc.store_scatter(outbuf, [baseq + ROT[d]], x)

        start_in(0, 0)
        start_in(1, 1)

        def block_loop(i, carry):
            s = lax.rem(i, 2)
            _dispatch(s, wait_in)
            pl.when(i >= 2)(lambda: _dispatch(s, wait_out))
            transpose_block(s)
            _dispatch(s, lambda sp: start_out(i, sp))
            pl.when(i + 2 < n_t)(
                lambda: _dispatch(s, lambda sp: start_in(i + 2, sp)))
            return carry

        lax.fori_loop(0, n_t, block_loop, 0)
        wait_out(0)
        wait_out(1)

        if TAIL:
            @pl.when(wid == EXTRA)
            def _():
                pltpu.sync_copy(wtail_hbm, tailbuf)
                pltpu.sync_copy(tailbuf,
                                wflat_hbm.at[pl.ds((V - TAIL) * E, TAIL * E)])

    return body


@functools.lru_cache(maxsize=None)
def _make_gather(V, E, L, B):
    info = plsc.get_sparse_core_info()
    NC, NS = info.num_cores, info.num_subcores
    NW = NC * NS
    CB = B // 128
    NBLK = L * CB
    PER_W = NBLK // NW
    EP = E + 1
    assert PER_W * NW == NBLK and PER_W % 2 == 0
    mesh = plsc.VectorSubcoreMesh(core_axis_name="c", subcore_axis_name="s")

    @functools.partial(
        pl.kernel,
        mesh=mesh,
        out_type=jax.ShapeDtypeStruct((L, E // 8, B // 128, 8, 128),
                                      jnp.float32),
        scratch_types=[
            pltpu.VMEM((2, 128), jnp.int32),
            pltpu.VMEM((2 * 128, E), jnp.float32),
            pltpu.VMEM((128, EP), jnp.float32),
            pltpu.VMEM((2, 8, 8, 128), jnp.float32),
            pltpu.SemaphoreType.DMA,
            pltpu.SemaphoreType.DMA,
            pltpu.SemaphoreType.DMA,
            pltpu.SemaphoreType.DMA,
            pltpu.SemaphoreType.DMA,
            pltpu.SemaphoreType.DMA,
        ],
        compiler_params=pltpu.CompilerParams(use_tc_tiling_on_sc=False,
                                             needs_layout_passes=False),
    )
    def body(w_hbm, idx_hbm, out_hbm, idxv, rowsv, skew, outb,
             mi0, mi1, mg0, mg1, mo0, mo1):
        wid = lax.axis_index("s") * NC + lax.axis_index("c")
        q0 = wid * PER_W
        mis = (mi0, mi1)
        mgs = (mg0, mg1)
        mos = (mo0, mo1)
        iota = _iota16()
        vb = [16 * j + iota for j in range(8)]

        def start_idx(i, s):
            q = q0 + i
            off = lax.div(q, CB) * B + lax.rem(q, CB) * 128
            pltpu.async_copy(idx_hbm.at[pl.ds(off, 128)], idxv.at[s], mis[s])

        def wait_idx(s):
            pltpu.make_async_copy(idx_hbm.at[pl.ds(0, 128)], idxv.at[s],
                                  mis[s]).wait()

        def start_gather(s):
            pltpu.async_copy(w_hbm.at[idxv.at[s]],
                             rowsv.at[pl.ds(s * 128, 128)], mgs[s])

        def wait_gather(s):
            pltpu.make_async_copy(w_hbm.at[idxv.at[s]],
                                  rowsv.at[pl.ds(s * 128, 128)], mgs[s]).wait()

        def start_out(i, s):
            q = q0 + i
            l = lax.div(q, CB)
            c = lax.rem(q, CB)
            pltpu.async_copy(outb.at[s], out_hbm.at[l, :, c], mos[s])

        def wait_out(s):
            pltpu.make_async_copy(outb.at[s], out_hbm.at[0, :, 0],
                                  mos[s]).wait()

        def transpose_block(s):
            rbase = s * 128

            @plsc.parallel_loop(0, 128, step=2, unroll=2)
            def _(k0):
                for dk in range(2):
                    k = k0 + dk
                    for q in range(4):
                        x = rowsv[rbase + k, pl.ds(16 * q, 16)]
                        skew[k, pl.ds(16 * q, 16)] = x

            @plsc.parallel_loop(0, E, unroll=4)
            def _(e):
                esplat = jnp.full((16,), e, jnp.int32)
                r = lax.div(e, 8)
                e8 = lax.rem(e, 8)
                for j in range(8):
                    x = plsc.load_gather(skew, [vb[j], esplat])
                    outb[s, r, e8, pl.ds(j * 16, 16)] = x

        start_idx(0, 0)
        start_idx(1, 1)
        wait_idx(0)
        start_gather(0)
        wait_idx(1)
        start_gather(1)

        def block_loop(i, carry):
            s = lax.rem(i, 2)
            _dispatch(s, wait_gather)
            pl.when(i + 2 < PER_W)(
                lambda: _dispatch(s, lambda sp: start_idx(i + 2, sp)))
            pl.when(i >= 2)(lambda: _dispatch(s, wait_out))
            transpose_block(s)
            _dispatch(s, lambda sp: start_out(i, sp))
            def _refill(sp):
                wait_idx(sp)
                start_gather(sp)

            pl.when(i + 2 < PER_W)(lambda: _dispatch(s, _refill))
            return carry

        lax.fori_loop(0, PER_W, block_loop, 0)
        wait_out(0)
        wait_out(1)

    return body


def kernel(batch_word_indexes, word_embedding):
    B, L = batch_word_indexes.shape
    V, E = word_embedding.shape
    idx_t = jnp.transpose(batch_word_indexes)
    w_t = jnp.transpose(word_embedding)
    n_tail = V % 128
    w_tail = word_embedding[V - n_tail:, :].reshape(-1)
    w_flat, idx_flat = _make_relayout(V, E, L, B)(w_t, idx_t, w_tail)
    w2 = w_flat.reshape(V, E)
    out5 = _make_gather(V, E, L, B)(w2, idx_flat)
    return jnp.transpose(out5, (2, 4, 0, 1, 3)).reshape(B, L, E)
